# baseline (speedup 1.0000x reference)
import functools

import jax
import jax.numpy as jnp
from jax import lax
from jax.experimental import pallas as pl
from jax.experimental.pallas import tpu as pltpu
from jax.experimental.pallas import tpu_sc as plsc

_N = 100000
_D = 64
_OUT_ROWS = 3 * _N

_CHUNK = 128
_CHUNKS = 25
_TILE_ROWS = _CHUNK * _CHUNKS
_TILE_STRIDE = 3128
_LAST_BASE = _N - _TILE_ROWS


def _sc_body(port_tab, port_idx, tech_tab, tech_idx, ip_x, out,
             pidx_v, tidx_v, row_a, row_b, row_c, sem_a, sem_b, sem_c):
    info = plsc.get_sparse_core_info()
    nc = info.num_cores
    wid = lax.axis_index("s") * nc + lax.axis_index("c")
    base = jnp.minimum(wid * _TILE_STRIDE, _LAST_BASE)

    pltpu.sync_copy(port_idx.at[pl.ds(base, _TILE_ROWS)], pidx_v)
    pltpu.sync_copy(tech_idx.at[pl.ds(base, _TILE_ROWS)], tidx_v)

    def step(j, _):
        off = j * _CHUNK
        pltpu.async_copy(port_tab.at[pidx_v.at[pl.ds(off, _CHUNK)]],
                         row_a, sem_a).wait()
        pltpu.sync_copy(row_a, out.at[pl.ds(_N + base + off, _CHUNK)])
        pltpu.async_copy(tech_tab.at[tidx_v.at[pl.ds(off, _CHUNK)]],
                         row_b, sem_b).wait()
        pltpu.sync_copy(row_b, out.at[pl.ds(2 * _N + base + off, _CHUNK)])
        pltpu.async_copy(ip_x.at[pl.ds(base + off, _CHUNK)],
                         row_c, sem_c).wait()
        pltpu.sync_copy(row_c, out.at[pl.ds(base + off, _CHUNK)])
        return _

    lax.fori_loop(0, _CHUNKS, step, 0)


def _sc_assemble(port_table, port_idx, tech_table, tech_idx, ip_x):
    mesh = plsc.VectorSubcoreMesh(core_axis_name="c", subcore_axis_name="s")
    fn = functools.partial(
        pl.kernel,
        mesh=mesh,
        compiler_params=pltpu.CompilerParams(use_tc_tiling_on_sc=False),
        out_type=jax.ShapeDtypeStruct((_OUT_ROWS, _D), jnp.float32),
        scratch_types=[
            pltpu.VMEM((_TILE_ROWS,), jnp.int32),
            pltpu.VMEM((_TILE_ROWS,), jnp.int32),
            pltpu.VMEM((_CHUNK, _D), jnp.float32),
            pltpu.VMEM((_CHUNK, _D), jnp.float32),
            pltpu.VMEM((_CHUNK, _D), jnp.float32),
            pltpu.SemaphoreType.DMA,
            pltpu.SemaphoreType.DMA,
            pltpu.SemaphoreType.DMA,
        ],
    )(_sc_body)
    return fn(port_table, port_idx, tech_table, tech_idx, ip_x)


_BM = 1000


def _tc_matmul_body(ip_ref, w_ref, b_ref, o_ref):
    o_ref[...] = jnp.dot(ip_ref[...], w_ref[...],
                         preferred_element_type=jnp.float32) + b_ref[...]


def _tc_matmul(ip2, w2, b2):
    return pl.pallas_call(
        _tc_matmul_body,
        grid=(_N // 2 // _BM,),
        in_specs=[
            pl.BlockSpec((_BM, _D), lambda i: (i, 0)),
            pl.BlockSpec((_D, 128), lambda i: (0, 0)),
            pl.BlockSpec((1, 128), lambda i: (0, 0)),
        ],
        out_specs=pl.BlockSpec((_BM, 128), lambda i: (i, 0)),
        out_shape=jax.ShapeDtypeStruct((_N // 2, 128), jnp.float32),
        compiler_params=pltpu.CompilerParams(
            dimension_semantics=("arbitrary",),
        ),
    )(ip2, w2, b2)


def kernel(ip_feats, port_idx, tech_idx, W_ip, b_ip, port_table, tech_table):
    ip2 = ip_feats.reshape(_N // 2, 64)
    w2 = jnp.zeros((_D, 128), jnp.float32)
    w2 = w2.at[0:32, 0:_D].set(W_ip).at[32:_D, _D:128].set(W_ip)
    b2 = jnp.concatenate([b_ip, b_ip]).reshape(1, 128)
    ip_x = _tc_matmul(ip2, w2, b2).reshape(_N, _D)
    return _sc_assemble(port_table, port_idx.astype(jnp.int32),
                        tech_table, tech_idx.astype(jnp.int32), ip_x)
